# baseline (speedup 1.0000x reference)
import functools

import jax
import jax.numpy as jnp
from jax import lax
from jax.experimental import pallas as pl
from jax.experimental.pallas import tpu as pltpu
from jax.experimental.pallas import tpu_sc as plsc

_NC = 2
_NS = 16
_L = 16
_NW = _NC * _NS


@functools.lru_cache(maxsize=None)
def _build(B, T, V, D):
    N = B * T
    n_per_w = N // _NW
    CH = 32
    n_chunks = n_per_w // CH
    assert n_per_w * _NW == N and n_chunks * CH == n_per_w
    assert T % n_per_w == 0 and D % _L == 0
    w_per_row = T // n_per_w

    mesh = plsc.VectorSubcoreMesh(core_axis_name="c", subcore_axis_name="s")

    @functools.partial(
        pl.kernel,
        out_type=jax.ShapeDtypeStruct((N, D), jnp.float32),
        mesh=mesh,
        scratch_types=[
            pltpu.VMEM((n_per_w,), jnp.int32),
            pltpu.VMEM((2, CH, D), jnp.float32),
            pltpu.VMEM((2, CH, D), jnp.float32),
            pltpu.SemaphoreType.DMA((2,)),
            pltpu.SemaphoreType.DMA((2,)),
            pltpu.SemaphoreType.DMA((2,)),
        ],
    )
    def emb_kernel(idx_hbm, tok_hbm, pos_hbm, out_hbm,
                   idx_v, tokb, posb, gsem, psem, osem):
        wid = lax.axis_index("s") * _NC + lax.axis_index("c")
        base = wid * n_per_w
        t_base = lax.rem(wid, w_per_row) * n_per_w

        pltpu.sync_copy(idx_hbm.at[pl.ds(base, n_per_w)], idx_v)

        def gather_desc(cc, s):
            return pltpu.make_async_copy(
                tok_hbm.at[idx_v.at[pl.ds(cc * CH, CH)]], tokb.at[s], gsem.at[s])

        def pos_desc(cc, s):
            return pltpu.make_async_copy(
                pos_hbm.at[pl.ds(t_base + cc * CH, CH)], posb.at[s], psem.at[s])

        def out_desc(cc, s):
            return pltpu.make_async_copy(
                posb.at[s], out_hbm.at[pl.ds(base + cc * CH, CH)], osem.at[s])

        for s in range(2):
            gather_desc(s, s).start()
            pos_desc(s, s).start()

        @pl.loop(0, n_chunks, step=2)
        def _(c):
            for s in range(2):
                cc = c + s
                gather_desc(cc, s).wait()
                pos_desc(cc, s).wait()

                @pl.loop(0, CH)
                def _(r):
                    for j in range(D // _L):
                        sl = pl.ds(j * _L, _L)
                        posb[s, r, sl] = tokb[s, r, sl] + posb[s, r, sl]

                out_desc(cc, s).start()

                @pl.when(cc + 2 < n_chunks)
                def _():
                    gather_desc(cc + 2, s).start()

                out_desc(cc, s).wait()

                @pl.when(cc + 2 < n_chunks)
                def _():
                    pos_desc(cc + 2, s).start()

    return emb_kernel


def kernel(x, token_emb, pos_emb):
    B, T = x.shape
    V, D = token_emb.shape
    out = _build(B, T, V, D)(
        x.reshape(-1).astype(jnp.int32), token_emb, pos_emb)
    return out.reshape(B, T, D)


# speedup vs baseline: 1.0465x; 1.0465x over previous
import functools

import jax
import jax.numpy as jnp
from jax import lax
from jax.experimental import pallas as pl
from jax.experimental.pallas import tpu as pltpu
from jax.experimental.pallas import tpu_sc as plsc

_NC = 2
_NS = 16
_L = 16
_NW = _NC * _NS


@functools.lru_cache(maxsize=None)
def _build(B, T, V, D):
    N = B * T
    t_per_w = T // _NW
    CH = 16
    n_chunks = t_per_w // CH
    assert t_per_w * _NW == T and n_chunks * CH == t_per_w
    assert D % _L == 0 and t_per_w % 8 == 0

    mesh = plsc.VectorSubcoreMesh(core_axis_name="c", subcore_axis_name="s")

    @functools.partial(
        pl.kernel,
        out_type=jax.ShapeDtypeStruct((N, D), jnp.float32),
        mesh=mesh,
        scratch_types=[
            pltpu.VMEM((B, t_per_w), jnp.int32),
            pltpu.VMEM((2, B, CH, D), jnp.float32),
            pltpu.VMEM((2, CH, D), jnp.float32),
            pltpu.SemaphoreType.DMA((2,)),
            pltpu.SemaphoreType.DMA((2,)),
            pltpu.SemaphoreType.DMA((2,)),
        ],
    )
    def emb_kernel(idx_hbm, tok_hbm, pos_hbm, out_hbm,
                   idx_v, tokb, posb, gsem, psem, osem):
        wid = lax.axis_index("s") * _NC + lax.axis_index("c")
        t_base = wid * t_per_w

        for b in range(B):
            pltpu.sync_copy(idx_hbm.at[pl.ds(b * T + t_base, t_per_w)],
                            idx_v.at[b])

        def gather_desc(cc, s, b):
            return pltpu.make_async_copy(
                tok_hbm.at[idx_v.at[b, pl.ds(cc * CH, CH)]],
                tokb.at[s, b], gsem.at[s])

        def pos_desc(cc, s):
            return pltpu.make_async_copy(
                pos_hbm.at[pl.ds(t_base + cc * CH, CH)], posb.at[s], psem.at[s])

        def out_desc(cc, s, b):
            return pltpu.make_async_copy(
                tokb.at[s, b], out_hbm.at[pl.ds(b * T + t_base + cc * CH, CH)],
                osem.at[s])

        for s in range(2):
            pos_desc(s, s).start()
            for b in range(B):
                gather_desc(s, s, b).start()

        @pl.loop(0, n_chunks, step=2)
        def _(c):
            for s in range(2):
                cc = c + s
                pos_desc(cc, s).wait()
                for b in range(B):
                    gather_desc(cc, s, b).wait()

                @pl.loop(0, CH)
                def _(r):
                    for j in range(D // _L):
                        sl = pl.ds(j * _L, _L)
                        p = posb[s, r, sl]
                        for b in range(B):
                            tokb[s, b, r, sl] = tokb[s, b, r, sl] + p

                for b in range(B):
                    out_desc(cc, s, b).start()

                @pl.when(cc + 2 < n_chunks)
                def _():
                    pos_desc(cc + 2, s).start()

                for b in range(B):
                    out_desc(cc, s, b).wait()

                @pl.when(cc + 2 < n_chunks)
                def _():
                    for b in range(B):
                        gather_desc(cc + 2, s, b).start()

    return emb_kernel


def kernel(x, token_emb, pos_emb):
    B, T = x.shape
    V, D = token_emb.shape
    out = _build(B, T, V, D)(
        x.reshape(-1).astype(jnp.int32), token_emb, pos_emb)
    return out.reshape(B, T, D)


# speedup vs baseline: 1.9066x; 1.8218x over previous
import functools

import jax
import jax.numpy as jnp
from jax import lax
from jax.experimental import pallas as pl
from jax.experimental.pallas import tpu as pltpu
from jax.experimental.pallas import tpu_sc as plsc

_NC = 2
_NS = 16
_L = 16
_NW = _NC * _NS


@functools.lru_cache(maxsize=None)
def _build(B, T, V, D):
    N = B * T
    t_per_w = T // _NW
    CH = 8
    NS = 4
    n_chunks = t_per_w // CH
    assert t_per_w * _NW == T and n_chunks * CH == t_per_w
    assert D % _L == 0 and t_per_w % 8 == 0
    assert n_chunks % NS == 0 and n_chunks >= NS

    mesh = plsc.VectorSubcoreMesh(core_axis_name="c", subcore_axis_name="s")

    @functools.partial(
        pl.kernel,
        out_type=jax.ShapeDtypeStruct((N, D), jnp.float32),
        mesh=mesh,
        scratch_types=[
            pltpu.VMEM((B, t_per_w), jnp.int32),
            pltpu.VMEM((NS, B, CH, D), jnp.float32),
            pltpu.VMEM((NS, CH, D), jnp.float32),
            pltpu.SemaphoreType.DMA((NS,)),
            pltpu.SemaphoreType.DMA((NS,)),
            pltpu.SemaphoreType.DMA((NS,)),
        ],
    )
    def emb_kernel(idx_hbm, tok_hbm, pos_hbm, out_hbm,
                   idx_v, tokb, posb, gsem, psem, osem):
        wid = lax.axis_index("s") * _NC + lax.axis_index("c")
        t_base = wid * t_per_w

        for b in range(B):
            pltpu.sync_copy(idx_hbm.at[pl.ds(b * T + t_base, t_per_w)],
                            idx_v.at[b])

        def gather_desc(cc, s, b):
            return pltpu.make_async_copy(
                tok_hbm.at[idx_v.at[b, pl.ds(cc * CH, CH)]],
                tokb.at[s, b], gsem.at[s])

        def pos_desc(cc, s):
            return pltpu.make_async_copy(
                pos_hbm.at[pl.ds(t_base + cc * CH, CH)], posb.at[s], psem.at[s])

        def out_desc(cc, s, b):
            return pltpu.make_async_copy(
                tokb.at[s, b], out_hbm.at[pl.ds(b * T + t_base + cc * CH, CH)],
                osem.at[s])

        for s in range(2):
            pos_desc(s, s).start()
            for b in range(B):
                gather_desc(s, s, b).start()

        @pl.loop(0, n_chunks, step=NS)
        def _(c):
            for s in range(NS):
                cc = c + s
                pos_desc(cc, s).wait()
                for b in range(B):
                    gather_desc(cc, s, b).wait()

                @pl.loop(0, CH)
                def _(r):
                    for j in range(D // _L):
                        sl = pl.ds(j * _L, _L)
                        p = posb[s, r, sl]
                        for b in range(B):
                            tokb[s, b, r, sl] = tokb[s, b, r, sl] + p

                for b in range(B):
                    out_desc(cc, s, b).start()

                sp = (s + 2) % NS

                @pl.when(cc + 2 < n_chunks)
                def _():
                    @pl.when(cc >= 2)
                    def _():
                        for b in range(B):
                            out_desc(cc - 2, sp, b).wait()

                    pos_desc(cc + 2, sp).start()
                    for b in range(B):
                        gather_desc(cc + 2, sp, b).start()

        for k in range(n_chunks - NS, n_chunks):
            for b in range(B):
                out_desc(k, k % NS, b).wait()

    return emb_kernel


def kernel(x, token_emb, pos_emb):
    B, T = x.shape
    V, D = token_emb.shape
    out = _build(B, T, V, D)(
        x.reshape(-1).astype(jnp.int32), token_emb, pos_emb)
    return out.reshape(B, T, D)


# speedup vs baseline: 1.9077x; 1.0006x over previous
import functools

import jax
import jax.numpy as jnp
from jax import lax
from jax.experimental import pallas as pl
from jax.experimental.pallas import tpu as pltpu
from jax.experimental.pallas import tpu_sc as plsc

_NC = 2
_NS = 16
_L = 16
_NW = _NC * _NS
_CH = 8


@functools.lru_cache(maxsize=None)
def _build(B, T, V, D):
    N = B * T
    t_per_w = T // _NW
    CH = _CH
    NS = 4
    n_chunks = t_per_w // CH
    assert t_per_w * _NW == T and n_chunks * CH == t_per_w
    assert D % _L == 0 and t_per_w % 8 == 0
    assert n_chunks % NS == 0 and n_chunks >= NS

    mesh = plsc.VectorSubcoreMesh(core_axis_name="c", subcore_axis_name="s")

    @functools.partial(
        pl.kernel,
        out_type=jax.ShapeDtypeStruct((N, D), jnp.float32),
        mesh=mesh,
        scratch_types=[
            pltpu.VMEM((B * t_per_w,), jnp.int32),
            pltpu.VMEM((NS, B * CH, D), jnp.float32),
            pltpu.VMEM((NS, CH, D), jnp.float32),
            pltpu.SemaphoreType.DMA((NS,)),
            pltpu.SemaphoreType.DMA((NS,)),
            pltpu.SemaphoreType.DMA((NS,)),
        ],
    )
    def emb_kernel(idx_hbm, tok_hbm, pos_hbm, out_hbm,
                   idx_v, tokb, posb, gsem, psem, osem):
        wid = lax.axis_index("s") * _NC + lax.axis_index("c")
        t_base = wid * t_per_w

        pltpu.sync_copy(idx_hbm.at[pl.ds(wid * (B * t_per_w), B * t_per_w)],
                        idx_v)

        def gather_desc(cc, s):
            return pltpu.make_async_copy(
                tok_hbm.at[idx_v.at[pl.ds(cc * (B * CH), B * CH)]],
                tokb.at[s], gsem.at[s])

        def pos_desc(cc, s):
            return pltpu.make_async_copy(
                pos_hbm.at[pl.ds(t_base + cc * CH, CH)], posb.at[s], psem.at[s])

        def out_desc(cc, s, b):
            return pltpu.make_async_copy(
                tokb.at[s, pl.ds(b * CH, CH)],
                out_hbm.at[pl.ds(b * T + t_base + cc * CH, CH)],
                osem.at[s])

        for s in range(2):
            pos_desc(s, s).start()
            gather_desc(s, s).start()

        @pl.loop(0, n_chunks, step=NS)
        def _(c):
            for s in range(NS):
                cc = c + s
                pos_desc(cc, s).wait()
                gather_desc(cc, s).wait()

                @pl.loop(0, CH)
                def _(r):
                    for j in range(D // _L):
                        sl = pl.ds(j * _L, _L)
                        p = posb[s, r, sl]
                        for b in range(B):
                            tokb[s, b * CH + r, sl] = tokb[s, b * CH + r, sl] + p

                for b in range(B):
                    out_desc(cc, s, b).start()

                sp = (s + 2) % NS

                @pl.when(cc + 2 < n_chunks)
                def _():
                    @pl.when(cc >= 2)
                    def _():
                        for b in range(B):
                            out_desc(cc - 2, sp, b).wait()

                    pos_desc(cc + 2, sp).start()
                    gather_desc(cc + 2, sp).start()

        for k in range(n_chunks - NS, n_chunks):
            for b in range(B):
                out_desc(k, k % NS, b).wait()

    return emb_kernel


def kernel(x, token_emb, pos_emb):
    B, T = x.shape
    V, D = token_emb.shape
    t_per_w = T // _NW
    xp = (x.astype(jnp.int32)
           .reshape(B, _NW, t_per_w // _CH, _CH)
           .transpose(1, 2, 0, 3)
           .reshape(-1))
    out = _build(B, T, V, D)(xp, token_emb, pos_emb)
    return out.reshape(B, T, D)
